# baseline (speedup 1.0000x reference)
import functools

import jax
import jax.numpy as jnp
from jax import lax
from jax.experimental import pallas as pl
from jax.experimental.pallas import tpu as pltpu
from jax.experimental.pallas import tpu_sc as plsc

N = 10000
E = 640000
G = 64
D1 = 112
D2 = 64
NC, NS = 2, 16
NW = NC * NS
K = 80
CH = 250
SEG = 5
CPS = CH // SEG
NR = 10240
RPT = NR // NS


def _make_seg_accum(d):
    @functools.partial(
        pl.kernel,
        out_type=jax.ShapeDtypeStruct((NC, NR, d), jnp.float32),
        mesh=plsc.VectorSubcoreMesh(core_axis_name="c", subcore_axis_name="s",
                                    num_cores=NC, num_subcores=NS),
        scratch_types=[
            pltpu.VMEM((CPS, K), jnp.int32),
            pltpu.VMEM((CPS, K), jnp.int32),
            pltpu.VMEM((K, d), jnp.float32),
            pltpu.VMEM((K, d), jnp.float32),
            pltpu.VMEM((K, d), jnp.float32),
            pltpu.VMEM((K, d), jnp.float32),
            pltpu.VMEM((K, d), jnp.float32),
            pltpu.VMEM_SHARED((NR, d), jnp.float32),
            pltpu.SemaphoreType.DMA,
            pltpu.SemaphoreType.DMA,
            pltpu.SemaphoreType.DMA,
            pltpu.SemaphoreType.DMA,
            pltpu.SemaphoreType.DMA,
        ],
        compiler_params=pltpu.CompilerParams(use_tc_tiling_on_sc=False),
    )
    def seg_accum(xp_hbm, eidx_hbm, zero_hbm, out_hbm,
                  src_v, dst_v, rows0_v, rows1_v, rows2_v, rows3_v, rows4_v,
                  acc_sh, sem0, sem1, sem2, sem3, sem4):
        c = lax.axis_index("c")
        s = lax.axis_index("s")
        w = s * NC + c
        pltpu.sync_copy(zero_hbm.at[pl.ds(s * RPT, RPT)],
                        acc_sh.at[pl.ds(s * RPT, RPT)])
        plsc.subcore_barrier()

        NB = 5
        bufs = (rows0_v, rows1_v, rows2_v, rows3_v, rows4_v)
        sems = (sem0, sem1, sem2, sem3, sem4)
        dummy = xp_hbm.at[pl.ds(0, K)]

        def seg_loop(gseg, carry):
            pltpu.sync_copy(eidx_hbm.at[0, w, pl.ds(gseg * CPS, CPS)], src_v)
            pltpu.sync_copy(eidx_hbm.at[1, w, pl.ds(gseg * CPS, CPS)], dst_v)
            for b in range(NB):
                pltpu.async_copy(xp_hbm.at[src_v.at[b]], bufs[b], sems[b])

            def body(t, carry2):
                for b in range(NB):
                    j = NB * t + b
                    pltpu.make_async_copy(dummy, bufs[b], sems[b]).wait()
                    pltpu.sync_copy(bufs[b], acc_sh.at[dst_v.at[j]],
                                    add=True)

                    @pl.when(j + NB < CPS)
                    def _():
                        pltpu.async_copy(xp_hbm.at[src_v.at[j + NB]],
                                         bufs[b], sems[b])

                return carry2

            lax.fori_loop(0, CPS // NB, body, 0)
            return carry

        lax.fori_loop(0, SEG, seg_loop, 0)
        plsc.subcore_barrier()
        pltpu.sync_copy(acc_sh.at[pl.ds(s * RPT, RPT)],
                        out_hbm.at[c, pl.ds(s * RPT, RPT)])

    return seg_accum


RP = 2000
GP = N // RP
R2 = 1000
G2 = N // R2

_FIELD_SLOTS = [0, 40, 60, 76, 92, 93, 94, 95, 96, 97, 98, 100, 108]


def _prep_body(*refs):
    parts = [r[...] for r in refs[:12]]
    parts.append(jnp.ones((RP, 1), jnp.float32))
    parts.append(jnp.zeros((RP, 3), jnp.float32))
    refs[12][...] = jnp.concatenate(parts, axis=1)


def _xr_body(x_ref, w1r_ref, b1_ref, out_ref):
    out_ref[...] = jnp.dot(x_ref[...], w1r_ref[...],
                           preferred_element_type=jnp.float32) + b1_ref[...]


def _mlp_body(p_ref, xr_ref, w1l_ref, w2l_ref, w2r_ref, b2_ref, sel_ref,
              y_ref, z_ref, inv_ref):
    pa = p_ref[0] + p_ref[1]
    deg = jnp.sum(pa * sel_ref[...], axis=1, keepdims=True)
    inv = 1.0 / jnp.maximum(deg, 1.0)
    h = jnp.dot(pa * inv, w1l_ref[...], preferred_element_type=jnp.float32)
    h = jnp.maximum(h + xr_ref[...], 0.0)
    y_ref[...] = jnp.dot(h, w2l_ref[...], preferred_element_type=jnp.float32)
    z_ref[...] = jnp.dot(h, w2r_ref[...],
                         preferred_element_type=jnp.float32) + b2_ref[...]
    inv_ref[...] = jnp.broadcast_to(inv, (RP, 8))


def _pool_body(p2_ref, z_ref, inv_ref, batch_ref, out_ref):
    i = pl.program_id(0)

    @pl.when(i == 0)
    def _():
        out_ref[...] = jnp.full((G, D2), -jnp.inf, jnp.float32)

    h2 = (p2_ref[0] + p2_ref[1]) * inv_ref[:, 0:1] + z_ref[...]
    gid = batch_ref[...]
    gmin = batch_ref[0, 0]
    gmax = batch_ref[R2 - 1, 0]
    for g in range(G):
        @pl.when(jnp.logical_and(gmin <= g, g <= gmax))
        def _():
            mx = jnp.max(jnp.where(gid == g, h2, -jnp.inf), axis=0,
                         keepdims=True)
            out_ref[g:g + 1, :] = jnp.maximum(out_ref[g:g + 1, :], mx)


def kernel(text, value, tag, classes, rx, ry, width, height, top, left,
           focused, votes, edge_index, batch, W1_l, W1_r, b1, W2_l, W2_r, b2):
    f32 = jnp.float32
    fields = (text, value, tag, classes, rx, ry, width, height, top, left,
              focused, votes)
    x_aug = pl.pallas_call(
        _prep_body,
        grid=(GP,),
        in_specs=[pl.BlockSpec((RP, f.shape[1]), lambda i: (i, 0))
                  for f in fields],
        out_specs=pl.BlockSpec((RP, D1), lambda i: (i, 0)),
        out_shape=jax.ShapeDtypeStruct((N, D1), f32),
    )(*fields)

    eidx = edge_index.reshape(2, NW, CH, K)

    p1 = _make_seg_accum(D1)(x_aug, eidx, jnp.zeros((NR, D1), f32))

    w1r = jnp.pad(W1_r, ((0, 4), (0, 0)))
    full = lambda shape: pl.BlockSpec(shape, lambda i: tuple(0 for _ in shape))
    xr = pl.pallas_call(
        _xr_body,
        grid=(GP,),
        in_specs=[pl.BlockSpec((RP, D1), lambda i: (i, 0)),
                  full((D1, 128)), full((1, 128))],
        out_specs=pl.BlockSpec((RP, 128), lambda i: (i, 0)),
        out_shape=jax.ShapeDtypeStruct((N, 128), f32),
    )(x_aug, w1r, b1.reshape(1, 128))

    w1l = jnp.pad(W1_l, ((0, 4), (0, 0)))
    sel = jnp.zeros((1, D1), f32).at[0, 108].set(1.0)
    y, z, inv = pl.pallas_call(
        _mlp_body,
        grid=(GP,),
        in_specs=[
            pl.BlockSpec((NC, RP, D1), lambda i: (0, i, 0)),
            pl.BlockSpec((RP, 128), lambda i: (i, 0)),
            full((D1, 128)),
            full((128, D2)), full((128, D2)), full((1, D2)),
            full((1, D1)),
        ],
        out_specs=[
            pl.BlockSpec((RP, D2), lambda i: (i, 0)),
            pl.BlockSpec((RP, D2), lambda i: (i, 0)),
            pl.BlockSpec((RP, 8), lambda i: (i, 0)),
        ],
        out_shape=[
            jax.ShapeDtypeStruct((N, D2), f32),
            jax.ShapeDtypeStruct((N, D2), f32),
            jax.ShapeDtypeStruct((N, 8), f32),
        ],
    )(p1, xr, w1l, W2_l, W2_r, b2.reshape(1, D2), sel)

    p2 = _make_seg_accum(D2)(y, eidx, jnp.zeros((NR, D2), f32))

    out = pl.pallas_call(
        _pool_body,
        grid=(G2,),
        in_specs=[
            pl.BlockSpec((NC, R2, D2), lambda i: (0, i, 0)),
            pl.BlockSpec((R2, D2), lambda i: (i, 0)),
            pl.BlockSpec((R2, 8), lambda i: (i, 0)),
            pl.BlockSpec((R2, 1), lambda i: (i, 0)),
        ],
        out_specs=pl.BlockSpec((G, D2), lambda i: (0, 0)),
        out_shape=jax.ShapeDtypeStruct((G, D2), f32),
        compiler_params=pltpu.CompilerParams(
            dimension_semantics=("arbitrary",)),
    )(p2, z, inv, batch.reshape(N, 1))
    return out


# speedup vs baseline: 1.0025x; 1.0025x over previous
import functools

import jax
import jax.numpy as jnp
from jax import lax
from jax.experimental import pallas as pl
from jax.experimental.pallas import tpu as pltpu
from jax.experimental.pallas import tpu_sc as plsc

N = 10000
E = 640000
G = 64
D1 = 112
D2 = 64
NC, NS = 2, 16
NW = NC * NS
K = 80
CH = 250
SEG = 5
CPS = CH // SEG
NR = 10240
RPT = NR // NS


def _make_seg_accum(d):
    @functools.partial(
        pl.kernel,
        out_type=jax.ShapeDtypeStruct((NC, NR, d), jnp.float32),
        mesh=plsc.VectorSubcoreMesh(core_axis_name="c", subcore_axis_name="s",
                                    num_cores=NC, num_subcores=NS),
        scratch_types=[
            pltpu.VMEM((CPS, K), jnp.int32),
            pltpu.VMEM((CPS, K), jnp.int32),
            pltpu.VMEM((K, d), jnp.float32),
            pltpu.VMEM((K, d), jnp.float32),
            pltpu.VMEM((K, d), jnp.float32),
            pltpu.VMEM((K, d), jnp.float32),
            pltpu.VMEM((K, d), jnp.float32),
            pltpu.VMEM_SHARED((NR, d), jnp.float32),
            pltpu.SemaphoreType.DMA,
            pltpu.SemaphoreType.DMA,
            pltpu.SemaphoreType.DMA,
            pltpu.SemaphoreType.DMA,
            pltpu.SemaphoreType.DMA,
        ],
        compiler_params=pltpu.CompilerParams(use_tc_tiling_on_sc=False),
    )
    def seg_accum(xp_hbm, eidx_hbm, out_hbm,
                  src_v, dst_v, rows0_v, rows1_v, rows2_v, rows3_v, rows4_v,
                  acc_sh, sem0, sem1, sem2, sem3, sem4):
        c = lax.axis_index("c")
        s = lax.axis_index("s")
        w = s * NC + c

        def zrow(r, carry):
            for cc in range(d // 16):
                rows0_v[r, pl.ds(cc * 16, 16)] = jnp.zeros((16,), jnp.float32)
            return carry

        lax.fori_loop(0, K, zrow, 0)
        for q in range(RPT // K):
            pltpu.sync_copy(rows0_v, acc_sh.at[pl.ds(s * RPT + q * K, K)])
        plsc.subcore_barrier()

        NB = 5
        bufs = (rows0_v, rows1_v, rows2_v, rows3_v, rows4_v)
        sems = (sem0, sem1, sem2, sem3, sem4)
        dummy = xp_hbm.at[pl.ds(0, K)]

        def seg_loop(gseg, carry):
            pltpu.sync_copy(eidx_hbm.at[0, w, pl.ds(gseg * CPS, CPS)], src_v)
            pltpu.sync_copy(eidx_hbm.at[1, w, pl.ds(gseg * CPS, CPS)], dst_v)
            for b in range(NB):
                pltpu.async_copy(xp_hbm.at[src_v.at[b]], bufs[b], sems[b])

            def body(t, carry2):
                for b in range(NB):
                    j = NB * t + b
                    pltpu.make_async_copy(dummy, bufs[b], sems[b]).wait()
                    pltpu.sync_copy(bufs[b], acc_sh.at[dst_v.at[j]],
                                    add=True)

                    @pl.when(j + NB < CPS)
                    def _():
                        pltpu.async_copy(xp_hbm.at[src_v.at[j + NB]],
                                         bufs[b], sems[b])

                return carry2

            lax.fori_loop(0, CPS // NB, body, 0)
            return carry

        lax.fori_loop(0, SEG, seg_loop, 0)
        plsc.subcore_barrier()
        pltpu.sync_copy(acc_sh.at[pl.ds(s * RPT, RPT)],
                        out_hbm.at[c, pl.ds(s * RPT, RPT)])

    return seg_accum


RP = 2000
GP = N // RP
R2 = 2000
G2 = N // R2

_FIELD_SLOTS = [0, 40, 60, 76, 92, 93, 94, 95, 96, 97, 98, 100, 108]


def _prep_body(*refs):
    parts = [r[...] for r in refs[:12]]
    parts.append(jnp.ones((RP, 1), jnp.float32))
    parts.append(jnp.zeros((RP, 3), jnp.float32))
    refs[12][...] = jnp.concatenate(parts, axis=1)


def _xr_body(x_ref, w1r_ref, b1_ref, out_ref):
    out_ref[...] = jnp.dot(x_ref[...], w1r_ref[...],
                           preferred_element_type=jnp.float32) + b1_ref[...]


def _mlp_body(p_ref, xr_ref, w1l_ref, w2l_ref, w2r_ref, b2_ref, sel_ref,
              y_ref, z_ref, inv_ref):
    pa = p_ref[0] + p_ref[1]
    deg = jnp.sum(pa * sel_ref[...], axis=1, keepdims=True)
    inv = 1.0 / jnp.maximum(deg, 1.0)
    h = jnp.dot(pa * inv, w1l_ref[...], preferred_element_type=jnp.float32)
    h = jnp.maximum(h + xr_ref[...], 0.0)
    y_ref[...] = jnp.dot(h, w2l_ref[...], preferred_element_type=jnp.float32)
    z_ref[...] = jnp.dot(h, w2r_ref[...],
                         preferred_element_type=jnp.float32) + b2_ref[...]
    inv_ref[...] = jnp.broadcast_to(inv, (RP, 8))


def _pool_body(p2_ref, z_ref, inv_ref, batch_ref, out_ref):
    i = pl.program_id(0)

    @pl.when(i == 0)
    def _():
        out_ref[...] = jnp.full((G, D2), -jnp.inf, jnp.float32)

    h2 = (p2_ref[0] + p2_ref[1]) * inv_ref[:, 0:1] + z_ref[...]
    gid = batch_ref[...]
    gmin = batch_ref[0, 0]
    gmax = batch_ref[R2 - 1, 0]
    for g in range(G):
        @pl.when(jnp.logical_and(gmin <= g, g <= gmax))
        def _():
            mx = jnp.max(jnp.where(gid == g, h2, -jnp.inf), axis=0,
                         keepdims=True)
            out_ref[g:g + 1, :] = jnp.maximum(out_ref[g:g + 1, :], mx)


def kernel(text, value, tag, classes, rx, ry, width, height, top, left,
           focused, votes, edge_index, batch, W1_l, W1_r, b1, W2_l, W2_r, b2):
    f32 = jnp.float32
    fields = (text, value, tag, classes, rx, ry, width, height, top, left,
              focused, votes)
    x_aug = pl.pallas_call(
        _prep_body,
        grid=(GP,),
        in_specs=[pl.BlockSpec((RP, f.shape[1]), lambda i: (i, 0))
                  for f in fields],
        out_specs=pl.BlockSpec((RP, D1), lambda i: (i, 0)),
        out_shape=jax.ShapeDtypeStruct((N, D1), f32),
    )(*fields)

    eidx = edge_index.reshape(2, NW, CH, K)

    p1 = _make_seg_accum(D1)(x_aug, eidx)

    w1r = jnp.pad(W1_r, ((0, 4), (0, 0)))
    full = lambda shape: pl.BlockSpec(shape, lambda i: tuple(0 for _ in shape))
    xr = pl.pallas_call(
        _xr_body,
        grid=(GP,),
        in_specs=[pl.BlockSpec((RP, D1), lambda i: (i, 0)),
                  full((D1, 128)), full((1, 128))],
        out_specs=pl.BlockSpec((RP, 128), lambda i: (i, 0)),
        out_shape=jax.ShapeDtypeStruct((N, 128), f32),
    )(x_aug, w1r, b1.reshape(1, 128))

    w1l = jnp.pad(W1_l, ((0, 4), (0, 0)))
    sel = jnp.zeros((1, D1), f32).at[0, 108].set(1.0)
    y, z, inv = pl.pallas_call(
        _mlp_body,
        grid=(GP,),
        in_specs=[
            pl.BlockSpec((NC, RP, D1), lambda i: (0, i, 0)),
            pl.BlockSpec((RP, 128), lambda i: (i, 0)),
            full((D1, 128)),
            full((128, D2)), full((128, D2)), full((1, D2)),
            full((1, D1)),
        ],
        out_specs=[
            pl.BlockSpec((RP, D2), lambda i: (i, 0)),
            pl.BlockSpec((RP, D2), lambda i: (i, 0)),
            pl.BlockSpec((RP, 8), lambda i: (i, 0)),
        ],
        out_shape=[
            jax.ShapeDtypeStruct((N, D2), f32),
            jax.ShapeDtypeStruct((N, D2), f32),
            jax.ShapeDtypeStruct((N, 8), f32),
        ],
    )(p1, xr, w1l, W2_l, W2_r, b2.reshape(1, D2), sel)

    p2 = _make_seg_accum(D2)(y, eidx)

    out = pl.pallas_call(
        _pool_body,
        grid=(G2,),
        in_specs=[
            pl.BlockSpec((NC, R2, D2), lambda i: (0, i, 0)),
            pl.BlockSpec((R2, D2), lambda i: (i, 0)),
            pl.BlockSpec((R2, 8), lambda i: (i, 0)),
            pl.BlockSpec((R2, 1), lambda i: (i, 0)),
        ],
        out_specs=pl.BlockSpec((G, D2), lambda i: (0, 0)),
        out_shape=jax.ShapeDtypeStruct((G, D2), f32),
        compiler_params=pltpu.CompilerParams(
            dimension_semantics=("arbitrary",)),
    )(p2, z, inv, batch.reshape(N, 1))
    return out
